# baseline (speedup 1.0000x reference)
import functools

import jax
import jax.numpy as jnp
from jax import lax
from jax.experimental import pallas as pl
from jax.experimental.pallas import tpu as pltpu
from jax.experimental.pallas import tpu_sc as plsc

_NC, _NS = 2, 16
_NW = _NC * _NS
_CHUNK = 128
_NSLOT = 4


def _c_body(tok_ref, pos_ref, c_ref):
    pos = pos_ref[...]
    V = tok_ref.shape[0]
    S = pos.shape[0]
    for v in range(V):
        c_ref[pl.ds(v * S, S), :] = pos + tok_ref[v][None]


def _build_c(token_emb, pos_emb):
    V, D = token_emb.shape
    S = pos_emb.shape[0]
    return pl.pallas_call(
        _c_body,
        out_shape=jax.ShapeDtypeStruct((V * S, D), jnp.float32),
    )(token_emb, pos_emb)


def _sc_body(b_per_w, x_hbm, c_hbm, out_hbm, x_v, c_sh, *slots):
    rows = slots[:_NSLOT]
    gsems = slots[_NSLOT:2 * _NSLOT]
    wsems = slots[2 * _NSLOT:]

    S = x_hbm.shape[1]
    n_chunks = b_per_w * S // _CHUNK

    wid = lax.axis_index("s") * _NC + lax.axis_index("c")
    base_b = wid * b_per_w

    sid = lax.axis_index("s")
    tab_per_sub = c_hbm.shape[0] // _NS
    tab_src = c_hbm.at[pl.ds(sid * tab_per_sub, tab_per_sub)]
    tab_dst = c_sh.at[pl.ds(sid * tab_per_sub, tab_per_sub)]
    pltpu.async_copy(tab_src, tab_dst, gsems[0])
    pltpu.async_copy(x_hbm.at[pl.ds(base_b, b_per_w)], x_v, gsems[1])
    pltpu.make_async_copy(tab_src, tab_dst, gsems[0]).wait()
    pltpu.make_async_copy(x_hbm.at[pl.ds(base_b, b_per_w)], x_v,
                          gsems[1]).wait()
    plsc.subcore_barrier()

    iota = lax.iota(jnp.int32, 16)
    base = base_b * S

    def start_g(i, c):
        pltpu.async_copy(c_sh.at[x_v.at[c]], rows[i], gsems[i])

    def wait_g(i, c):
        pltpu.make_async_copy(c_sh.at[x_v.at[c]], rows[i], gsems[i]).wait()

    def start_w(i, c):
        pltpu.async_copy(rows[i], out_hbm.at[pl.ds(base + c * _CHUNK, _CHUNK)],
                         wsems[i])

    def wait_w(i, c):
        pltpu.make_async_copy(rows[i],
                              out_hbm.at[pl.ds(base + c * _CHUNK, _CHUNK)],
                              wsems[i]).wait()

    def round_body(r, carry):
        for i in range(_NSLOT):
            c = r * _NSLOT + i
            for k in range(S // 16):
                sl = pl.ds(k * 16, 16)
                x_v[c, sl] = x_v[c, sl] * S + (iota + k * 16)

            @pl.when(r > 0)
            def _drain():
                wait_w(i, c)

            start_g(i, c)
        for i in range(_NSLOT):
            c = r * _NSLOT + i
            wait_g(i, c)
            start_w(i, c)
        return carry

    lax.fori_loop(0, n_chunks // _NSLOT, round_body, 0)
    for i in range(_NSLOT):
        wait_w(i, 0)


def kernel(x, token_emb, pos_emb):
    x = x.astype(jnp.int32)
    B, S = x.shape
    V, D = token_emb.shape
    c_tab = _build_c(token_emb, pos_emb)

    b_per_w = B // _NW

    mesh = plsc.VectorSubcoreMesh(core_axis_name="c", subcore_axis_name="s",
                                  num_cores=_NC, num_subcores=_NS)
    body = functools.partial(_sc_body, b_per_w)
    out = pl.kernel(
        body,
        out_type=jax.ShapeDtypeStruct((B * S, D), jnp.float32),
        mesh=mesh,
        scratch_types=[
            pltpu.VMEM((b_per_w, S), jnp.int32),
            pltpu.VMEM_SHARED((V * S, D), jnp.float32),
        ] + [pltpu.VMEM((_CHUNK, D), jnp.float32)] * _NSLOT
          + [pltpu.SemaphoreType.DMA] * (2 * _NSLOT),
    )(x, c_tab)
    return out.reshape(B, S, D)


# speedup vs baseline: 1.0102x; 1.0102x over previous
import functools

import jax
import jax.numpy as jnp
from jax import lax
from jax.experimental import pallas as pl
from jax.experimental.pallas import tpu as pltpu
from jax.experimental.pallas import tpu_sc as plsc

_NC, _NS = 2, 16
_NW = _NC * _NS
_CHUNK = 64
_NBUF = 12


def _c_body(tok_ref, pos_ref, c_ref):
    pos = pos_ref[...]
    V = tok_ref.shape[0]
    S = pos.shape[0]
    for v in range(V):
        c_ref[pl.ds(v * S, S), :] = pos + tok_ref[v][None]


def _build_c(token_emb, pos_emb):
    V, D = token_emb.shape
    S = pos_emb.shape[0]
    return pl.pallas_call(
        _c_body,
        out_shape=jax.ShapeDtypeStruct((V * S, D), jnp.float32),
    )(token_emb, pos_emb)


def _sc_body(b_per_w, x_hbm, c_hbm, out_hbm, x_v, c_sh, *slots):
    rows = slots[:_NBUF]
    gsems = slots[_NBUF:2 * _NBUF]
    wsems = slots[2 * _NBUF:]

    S = x_hbm.shape[1]
    n_chunks = b_per_w * S // _CHUNK
    per_row = S // _CHUNK

    wid = lax.axis_index("s") * _NC + lax.axis_index("c")
    base_b = wid * b_per_w

    sid = lax.axis_index("s")
    tab_per_sub = c_hbm.shape[0] // _NS
    tab_src = c_hbm.at[pl.ds(sid * tab_per_sub, tab_per_sub)]
    tab_dst = c_sh.at[pl.ds(sid * tab_per_sub, tab_per_sub)]
    pltpu.async_copy(tab_src, tab_dst, gsems[0])
    pltpu.async_copy(x_hbm.at[pl.ds(base_b, b_per_w)], x_v, gsems[1])
    pltpu.make_async_copy(tab_src, tab_dst, gsems[0]).wait()
    pltpu.make_async_copy(x_hbm.at[pl.ds(base_b, b_per_w)], x_v,
                          gsems[1]).wait()
    plsc.subcore_barrier()

    iota = lax.iota(jnp.int32, 16)
    base = base_b * S

    def compute_idx(c):
        r = lax.div(c, per_row)
        s0 = lax.rem(c, per_row) * _CHUNK
        for k in range(_CHUNK // 16):
            sl = pl.ds(s0 + k * 16, 16)
            x_v[r, sl] = x_v[r, sl] * S + (iota + s0 + k * 16)

    def idx_view(c):
        return x_v.at[lax.div(c, per_row), pl.ds(lax.rem(c, per_row) * _CHUNK,
                                                 _CHUNK)]

    def start_g(i, c):
        pltpu.async_copy(c_sh.at[idx_view(c)], rows[i], gsems[i])

    def wait_g(i, c):
        pltpu.make_async_copy(c_sh.at[idx_view(c)], rows[i], gsems[i]).wait()

    def start_w(i, c):
        pltpu.async_copy(rows[i], out_hbm.at[pl.ds(base + c * _CHUNK, _CHUNK)],
                         wsems[i])

    def wait_w(i, c):
        pltpu.make_async_copy(rows[i],
                              out_hbm.at[pl.ds(base + c * _CHUNK, _CHUNK)],
                              wsems[i]).wait()

    n_iters = n_chunks // _NBUF

    def round_body(rr, carry):
        c0 = rr * _NBUF
        for i in range(_NBUF):
            c = c0 + i
            compute_idx(c)

            @pl.when(rr > 0)
            def _drain():
                wait_w(i, c)

            start_g(i, c)
        for i in range(_NBUF):
            c = c0 + i
            wait_g(i, c)
            start_w(i, c)
        return carry

    lax.fori_loop(0, n_iters, round_body, 0)
    for j, c in enumerate(range(n_iters * _NBUF, n_chunks)):
        compute_idx(c)
        wait_w(j, c)
        start_g(j, c)
        wait_g(j, c)
        start_w(j, c)
    for i in range(_NBUF):
        wait_w(i, 0)


def kernel(x, token_emb, pos_emb):
    x = x.astype(jnp.int32)
    B, S = x.shape
    V, D = token_emb.shape
    c_tab = _build_c(token_emb, pos_emb)

    b_per_w = B // _NW

    mesh = plsc.VectorSubcoreMesh(core_axis_name="c", subcore_axis_name="s",
                                  num_cores=_NC, num_subcores=_NS)
    body = functools.partial(_sc_body, b_per_w)
    out = pl.kernel(
        body,
        out_type=jax.ShapeDtypeStruct((B * S, D), jnp.float32),
        mesh=mesh,
        scratch_types=[
            pltpu.VMEM((b_per_w, S), jnp.int32),
            pltpu.VMEM_SHARED((V * S, D), jnp.float32),
        ] + [pltpu.VMEM((_CHUNK, D), jnp.float32)] * _NBUF
          + [pltpu.SemaphoreType.DMA] * (2 * _NBUF),
    )(x, c_tab)
    return out.reshape(B, S, D)
